# baseline (speedup 1.0000x reference)
import functools

import jax
import jax.numpy as jnp
from jax import lax
from jax.experimental import pallas as pl
from jax.experimental.pallas import tpu as pltpu
from jax.experimental.pallas import tpu_sc as plsc

_NW = 32


def _gather_pipeline(jobs, idx_v, bufs, gsems, osems, depth=4):
    k = len(jobs)
    gcps = [None] * k
    ocps = [None] * k

    def fire(j):
        tab, jj, _ = jobs[j]
        b = j % depth
        gcps[j] = pltpu.async_copy(tab.at[idx_v.at[jj]], bufs.at[b],
                                   gsems.at[b])

    for j in range(min(depth, k)):
        fire(j)
    for j in range(k):
        b = j % depth
        gcps[j].wait()
        ocps[j] = pltpu.async_copy(bufs.at[b], jobs[j][2], osems.at[b])
        if j + depth < k:
            ocps[j].wait()
            fire(j + depth)
    for j in range(max(0, k - depth), k):
        ocps[j].wait()


def _sc_gather(table, idx3d, n_out, k_chunks):
    rpw = k_chunks
    mesh = plsc.VectorSubcoreMesh(core_axis_name="c", subcore_axis_name="s")

    @functools.partial(
        pl.kernel, mesh=mesh,
        out_type=jax.ShapeDtypeStruct((n_out, 128), jnp.float32),
        scratch_types=[
            pltpu.VMEM((rpw, 128), jnp.int32),
            pltpu.VMEM((4, 128, 128), jnp.float32),
            pltpu.SemaphoreType.DMA((4,)),
            pltpu.SemaphoreType.DMA((4,)),
        ],
    )
    def k(table_hbm, idx_hbm, out_hbm, idx_v, bufs, gsems, osems):
        wid = lax.axis_index("s") * 2 + lax.axis_index("c")
        pltpu.sync_copy(idx_hbm.at[wid], idx_v)
        jobs = [(table_hbm, j,
                 out_hbm.at[pl.ds((wid * k_chunks + j) * 128, 128)])
                for j in range(k_chunks)]
        _gather_pipeline(jobs, idx_v, bufs, gsems, osems)

    return k(table, idx3d)


def _sc_gather2(table_a, table_b, idx3d, n_out, k_chunks):
    rpw = k_chunks
    mesh = plsc.VectorSubcoreMesh(core_axis_name="c", subcore_axis_name="s")

    @functools.partial(
        pl.kernel, mesh=mesh,
        out_type=(jax.ShapeDtypeStruct((n_out, 128), jnp.float32),
                  jax.ShapeDtypeStruct((n_out, 128), jnp.float32)),
        scratch_types=[
            pltpu.VMEM((rpw, 128), jnp.int32),
            pltpu.VMEM((4, 128, 128), jnp.float32),
            pltpu.SemaphoreType.DMA((4,)),
            pltpu.SemaphoreType.DMA((4,)),
        ],
    )
    def k(ta_hbm, tb_hbm, idx_hbm, oa_hbm, ob_hbm, idx_v, bufs,
          gsems, osems):
        wid = lax.axis_index("s") * 2 + lax.axis_index("c")
        pltpu.sync_copy(idx_hbm.at[wid], idx_v)
        jobs = []
        for tab, out in ((ta_hbm, oa_hbm), (tb_hbm, ob_hbm)):
            jobs += [(tab, j,
                      out.at[pl.ds((wid * k_chunks + j) * 128, 128)])
                     for j in range(k_chunks)]
        _gather_pipeline(jobs, idx_v, bufs, gsems, osems)

    return k(table_a, table_b, idx3d)


def _sc_scatter(msg128, idx3d, zeros_pk):
    n_pk = zeros_pk.shape[0]
    e_pad = msg128.shape[0]
    rpw = e_pad // _NW // 128
    npt = n_pk // 16
    mesh = plsc.VectorSubcoreMesh(core_axis_name="c", subcore_axis_name="s")

    @functools.partial(
        pl.kernel, mesh=mesh,
        out_type=jax.ShapeDtypeStruct((2, n_pk, 128), jnp.float32),
        scratch_types=[
            pltpu.VMEM((rpw, 128), jnp.int32),
            pltpu.VMEM((2, 128, 128), jnp.float32),
            pltpu.SemaphoreType.DMA((2,)),
            pltpu.VMEM_SHARED((n_pk, 128), jnp.float32),
        ],
    )
    def k(msg_hbm, idx_hbm, zero_hbm, out_hbm, idx_v, bufs, lsems, agg_sh):
        c = lax.axis_index("c")
        s = lax.axis_index("s")
        wid = s * 2 + c
        pltpu.sync_copy(zero_hbm.at[pl.ds(s * npt, npt)],
                        agg_sh.at[pl.ds(s * npt, npt)])
        pltpu.sync_copy(idx_hbm.at[wid], idx_v)
        plsc.subcore_barrier()
        lcps = [None] * rpw

        def fire(j):
            lcps[j] = pltpu.async_copy(
                msg_hbm.at[pl.ds((wid * rpw + j) * 128, 128)],
                bufs.at[j & 1], lsems.at[j & 1])

        fire(0)
        for j in range(rpw):
            lcps[j].wait()
            if j + 1 < rpw:
                fire(j + 1)
            pltpu.sync_copy(bufs.at[j & 1], agg_sh.at[idx_v.at[j]],
                            add=True)
        plsc.subcore_barrier()
        pltpu.sync_copy(agg_sh.at[pl.ds(s * npt, npt)],
                        out_hbm.at[c, pl.ds(s * npt, npt)])

    return k(msg128, idx3d, zeros_pk)


def _sc_scatter2(msg_a, msg_b, idx3d, zeros_pk):
    n_pk = zeros_pk.shape[0]
    e_pad = msg_a.shape[0]
    rpw = e_pad // _NW // 128
    npt = n_pk // 16
    mesh = plsc.VectorSubcoreMesh(core_axis_name="c", subcore_axis_name="s")

    @functools.partial(
        pl.kernel, mesh=mesh,
        out_type=jax.ShapeDtypeStruct((2, 2, n_pk, 128), jnp.float32),
        scratch_types=[
            pltpu.VMEM((rpw, 128), jnp.int32),
            pltpu.VMEM((2, 128, 128), jnp.float32),
            pltpu.SemaphoreType.DMA((2,)),
            pltpu.VMEM_SHARED((n_pk, 128), jnp.float32),
            pltpu.VMEM_SHARED((n_pk, 128), jnp.float32),
        ],
    )
    def k(ma_hbm, mb_hbm, idx_hbm, zero_hbm, out_hbm, idx_v, bufs,
          lsems, agg_a, agg_b):
        c = lax.axis_index("c")
        s = lax.axis_index("s")
        wid = s * 2 + c
        pltpu.sync_copy(zero_hbm.at[pl.ds(s * npt, npt)],
                        agg_a.at[pl.ds(s * npt, npt)])
        pltpu.sync_copy(zero_hbm.at[pl.ds(s * npt, npt)],
                        agg_b.at[pl.ds(s * npt, npt)])
        pltpu.sync_copy(idx_hbm.at[wid], idx_v)
        plsc.subcore_barrier()
        jobs = []
        for msg, acc in ((ma_hbm, agg_a), (mb_hbm, agg_b)):
            jobs += [(msg.at[pl.ds((wid * rpw + j) * 128, 128)], acc, j)
                     for j in range(rpw)]
        lcps = [None] * len(jobs)

        def fire(j):
            lcps[j] = pltpu.async_copy(jobs[j][0], bufs.at[j & 1],
                                       lsems.at[j & 1])

        fire(0)
        for j in range(len(jobs)):
            lcps[j].wait()
            if j + 1 < len(jobs):
                fire(j + 1)
            pltpu.sync_copy(bufs.at[j & 1],
                            jobs[j][1].at[idx_v.at[jobs[j][2]]], add=True)
        plsc.subcore_barrier()
        pltpu.sync_copy(agg_a.at[pl.ds(s * npt, npt)],
                        out_hbm.at[c, 0, pl.ds(s * npt, npt)])
        pltpu.sync_copy(agg_b.at[pl.ds(s * npt, npt)],
                        out_hbm.at[c, 1, pl.ds(s * npt, npt)])

    return k(msg_a, msg_b, idx3d, zeros_pk)

_DIM = 32
_EB = 1024
_NB = 2048
_TB = 400
_S2S_C = 2000


def _lin0_body(x_ref, w_ref, b_ref, o_ref):
    res = jnp.maximum(
        jnp.dot(x_ref[...], w_ref[...], preferred_element_type=jnp.float32)
        + b_ref[...], 0.0)
    o_ref[...] = jnp.concatenate(
        [res, jnp.zeros((res.shape[0], 128 - _DIM), jnp.float32)], axis=1)


def _lin0(x, w, b):
    n = x.shape[0]
    return pl.pallas_call(
        _lin0_body,
        grid=(n // _NB,),
        in_specs=[
            pl.BlockSpec((_NB, 3), lambda i: (i, 0)),
            pl.BlockSpec((3, _DIM), lambda i: (0, 0)),
            pl.BlockSpec((1, _DIM), lambda i: (0, 0)),
        ],
        out_specs=pl.BlockSpec((_NB, 128), lambda i: (i, 0)),
        out_shape=jax.ShapeDtypeStruct((n, 128), jnp.float32),
    )(x, w, b[None, :])


def _edgenet_body(ea_ref, w1_ref, b1_ref, w2_ref, b2_ref, o_ref):
    h1 = jnp.maximum(
        jnp.dot(w1_ref[...], ea_ref[...], preferred_element_type=jnp.float32)
        + b1_ref[...], 0.0)
    o_ref[...] = (
        jnp.dot(w2_ref[...], h1, preferred_element_type=jnp.float32)
        + b2_ref[...])


def _edgenet(ea_t, nn1_W, nn1_b, nn2_W, nn2_b):
    e_pad = ea_t.shape[1]
    c = _DIM * _DIM
    return pl.pallas_call(
        _edgenet_body,
        grid=(e_pad // _EB,),
        in_specs=[
            pl.BlockSpec((7, _EB), lambda i: (0, i)),
            pl.BlockSpec((128, 7), lambda i: (0, 0)),
            pl.BlockSpec((128, 1), lambda i: (0, 0)),
            pl.BlockSpec((c, 128), lambda i: (0, 0)),
            pl.BlockSpec((c, 1), lambda i: (0, 0)),
        ],
        out_specs=pl.BlockSpec((c, _EB), lambda i: (0, i)),
        out_shape=jax.ShapeDtypeStruct((c, e_pad), jnp.float32),
        compiler_params=pltpu.CompilerParams(
            dimension_semantics=("arbitrary",)),
    )(ea_t, nn1_W.T, nn1_b[:, None], nn2_W.T, nn2_b[:, None])


def _einsum_body(we_ref, a_ref, qm_ref, o_ref):
    a = a_ref[...]
    acc = we_ref[pl.ds(0, _DIM), :] * a[0:1, :]
    for d in range(1, _DIM):
        acc = acc + we_ref[pl.ds(d * _DIM, _DIM), :] * a[d:d + 1, :]
    qm = qm_ref[...]
    o_ref[...] = jnp.concatenate(
        [acc * qm[q:q + 1, :] for q in range(4)], axis=0)


def _edge_einsum(we_c, a_t, qm4):
    e_pad = we_c.shape[1]
    c = _DIM * _DIM
    return pl.pallas_call(
        _einsum_body,
        grid=(e_pad // _EB,),
        in_specs=[
            pl.BlockSpec((c, _EB), lambda i: (0, i)),
            pl.BlockSpec((_DIM, _EB), lambda i: (0, i)),
            pl.BlockSpec((4, _EB), lambda i: (0, i)),
        ],
        out_specs=pl.BlockSpec((128, _EB), lambda i: (0, i)),
        out_shape=jax.ShapeDtypeStruct((128, e_pad), jnp.float32),
        compiler_params=pltpu.CompilerParams(
            dimension_semantics=("arbitrary",)),
    )(we_c, a_t, qm4)


def _node_body(out_ref, agg_ref, deg_ref, wc_ref, bc_ref,
               wir_ref, wiz_ref, win_ref, whr_ref, whz_ref, whn_ref,
               o_ref):
    out = out_ref[...][:, :_DIM]
    aggn = agg_ref[...] / jnp.maximum(deg_ref[...], 1.0)
    dot = functools.partial(jnp.dot, preferred_element_type=jnp.float32)
    m = jnp.maximum(dot(out, wc_ref[...]) + aggn + bc_ref[...], 0.0)
    r = jax.nn.sigmoid(dot(m, wir_ref[...]) + dot(out, whr_ref[...]))
    z = jax.nn.sigmoid(dot(m, wiz_ref[...]) + dot(out, whz_ref[...]))
    n = jnp.tanh(dot(m, win_ref[...]) + r * dot(out, whn_ref[...]))
    new = (1.0 - z) * n + z * out
    o_ref[...] = jnp.concatenate(
        [new, jnp.zeros((new.shape[0], 128 - _DIM), jnp.float32)], axis=1)


def _node_update(out, agg, deg32, p):
    n = out.shape[0]
    wih = p['gru_Wih']
    whh = p['gru_Whh']
    nb128 = pl.BlockSpec((_NB, 128), lambda i: (i, 0))
    nb = pl.BlockSpec((_NB, _DIM), lambda i: (i, 0))
    wb = pl.BlockSpec((_DIM, _DIM), lambda i: (0, 0))
    return pl.pallas_call(
        _node_body,
        grid=(n // _NB,),
        in_specs=[nb128, nb, nb, wb,
                  pl.BlockSpec((1, _DIM), lambda i: (0, 0)),
                  wb, wb, wb, wb, wb, wb],
        out_specs=nb128,
        out_shape=jax.ShapeDtypeStruct((n, 128), jnp.float32),
    )(out, agg, deg32, p['conv_root'], p['conv_b'][None, :],
      wih[:, :_DIM], wih[:, _DIM:2 * _DIM], wih[:, 2 * _DIM:],
      whh[:, :_DIM], whh[:, _DIM:2 * _DIM], whh[:, 2 * _DIM:])


def _s2s_loop(out_ref, wih_ref, whh_ref, bih_ref, bhh_ref, n_real):
    dot = functools.partial(jnp.dot, preferred_element_type=jnp.float32)
    nch = n_real // _S2S_C
    q_star = jnp.zeros((1, 2 * _DIM), jnp.float32)
    hs = jnp.zeros((1, _DIM), jnp.float32)
    cs = jnp.zeros((1, _DIM), jnp.float32)
    for _ in range(6):
        g = (dot(q_star, wih_ref[...]) + bih_ref[...]
             + dot(hs, whh_ref[...]) + bhh_ref[...])
        i = jax.nn.sigmoid(g[:, :_DIM])
        f = jax.nn.sigmoid(g[:, _DIM:2 * _DIM])
        gg = jnp.tanh(g[:, 2 * _DIM:3 * _DIM])
        o = jax.nn.sigmoid(g[:, 3 * _DIM:])
        cs = f * cs + i * gg
        hs = o * jnp.tanh(cs)

        def _emax_body(k, m):
            ch = out_ref[pl.ds(k * _S2S_C, _S2S_C), :][:, :_DIM]
            e = jnp.sum(ch * hs, axis=1, keepdims=True)
            return jnp.maximum(m, jnp.max(e))

        emax = lax.fori_loop(0, nch, _emax_body, jnp.float32(-jnp.inf))

        def _acc_body(k, carry):
            den, racc = carry
            ch = out_ref[pl.ds(k * _S2S_C, _S2S_C), :][:, :_DIM]
            e = jnp.sum(ch * hs, axis=1, keepdims=True)
            ex = jnp.exp(e - emax)
            return (den + jnp.sum(ex),
                    racc + jnp.sum(ex * ch, axis=0, keepdims=True))

        den, racc = lax.fori_loop(
            0, nch, _acc_body,
            (jnp.float32(0.0), jnp.zeros((1, _DIM), jnp.float32)))
        q_star = jnp.concatenate([hs, racc / den], axis=1)
    return q_star


def _s2s_actor(out, p, n_real):
    def body(out_ref, wih_ref, whh_ref, bih_ref, bhh_ref, q_ref):
        q_ref[...] = _s2s_loop(out_ref, wih_ref, whh_ref, bih_ref,
                               bhh_ref, n_real)

    return pl.pallas_call(
        body,
        out_shape=jax.ShapeDtypeStruct((1, 2 * _DIM), jnp.float32),
    )(out, p['s2s_Wih'], p['s2s_Whh'], p['s2s_bih'][None, :],
      p['s2s_bhh'][None, :])


def _s2s_critic_body(n_real, out_ref, wih_ref, whh_ref, bih_ref, bhh_ref,
                     mwih_ref, mb_ref, l1_ref, l1b_ref, l3_ref, l3b_ref,
                     v_ref):
    q_star = _s2s_loop(out_ref, wih_ref, whh_ref, bih_ref, bhh_ref, n_real)
    dot = functools.partial(jnp.dot, preferred_element_type=jnp.float32)
    hid = 2 * _DIM
    g = dot(q_star, mwih_ref[...]) + mb_ref[...]
    i = jax.nn.sigmoid(g[:, :hid])
    gg = jnp.tanh(g[:, 2 * hid:3 * hid])
    o = jax.nn.sigmoid(g[:, 3 * hid:])
    hv = o * jnp.tanh(i * gg)
    oc = jnp.maximum(dot(hv, l1_ref[...]) + l1b_ref[...], 0.0)
    v_ref[...] = dot(oc, l3_ref[...]) + l3b_ref[...]


def _s2s_critic(out, p, n_real):
    return pl.pallas_call(
        functools.partial(_s2s_critic_body, n_real),
        out_shape=jax.ShapeDtypeStruct((1, 1), jnp.float32),
    )(out, p['s2s_Wih'], p['s2s_Whh'], p['s2s_bih'][None, :],
      p['s2s_bhh'][None, :],
      p['mem_Wih'], (p['mem_bih'] + p['mem_bhh'])[None, :],
      p['lin1_W'], p['lin1_b'][None, :], p['lin3_W'], p['lin3_b'][None, :])


def _head_body(feat_ref, wi_ref, wg_ref, wo_ref, bi_ref, bg_ref, bo_ref,
               l1_ref, l1b_ref, l2_ref, l2b_ref,
               logits_ref, logp_ref, ent_ref):
    dot = functools.partial(jnp.dot, preferred_element_type=jnp.float32)
    feat = feat_ref[...]
    i = jax.nn.sigmoid(dot(feat, wi_ref[...]) + bi_ref[...])
    gg = jnp.tanh(dot(feat, wg_ref[...]) + bg_ref[...])
    o = jax.nn.sigmoid(dot(feat, wo_ref[...]) + bo_ref[...])
    hm = o * jnp.tanh(i * gg)
    o1 = jnp.maximum(dot(hm, l1_ref[...]) + l1b_ref[...], 0.0)
    logits = dot(o1, l2_ref[...]) + l2b_ref[...]
    m = jnp.max(logits, axis=1, keepdims=True)
    ex = jnp.exp(logits - m)
    lse = jnp.log(jnp.sum(ex, axis=1, keepdims=True)) + m
    logp = logits - lse
    logits_ref[...] = logits
    logp_ref[...] = logp
    ent_ref[...] = -jnp.sum(jnp.exp(logp) * logp, axis=1, keepdims=True)


def _actor_head(feat, p, action_dim):
    t = feat.shape[0]
    hid = 6 * _DIM
    wih = p['mem_Wih']
    b = p['mem_bih'] + p['mem_bhh']
    fb = pl.BlockSpec((_TB, hid), lambda i: (i, 0))
    wb = pl.BlockSpec((hid, hid), lambda i: (0, 0))
    bb = pl.BlockSpec((1, hid), lambda i: (0, 0))
    ob = pl.BlockSpec((_TB, action_dim), lambda i: (i, 0))
    return pl.pallas_call(
        _head_body,
        grid=(t // _TB,),
        in_specs=[fb, wb, wb, wb, bb, bb, bb,
                  pl.BlockSpec((hid, 2 * _DIM), lambda i: (0, 0)),
                  pl.BlockSpec((1, 2 * _DIM), lambda i: (0, 0)),
                  pl.BlockSpec((2 * _DIM, action_dim), lambda i: (0, 0)),
                  pl.BlockSpec((1, action_dim), lambda i: (0, 0))],
        out_specs=(ob, ob, pl.BlockSpec((_TB, 1), lambda i: (i, 0))),
        out_shape=(
            jax.ShapeDtypeStruct((t, action_dim), jnp.float32),
            jax.ShapeDtypeStruct((t, action_dim), jnp.float32),
            jax.ShapeDtypeStruct((t, 1), jnp.float32),
        ),
    )(feat, wih[:, :hid], wih[:, 2 * hid:3 * hid], wih[:, 3 * hid:],
      b[None, :hid], b[None, 2 * hid:3 * hid], b[None, 3 * hid:],
      p['lin1_W'], p['lin1_b'][None, :], p['lin2_W'], p['lin2_b'][None, :])


def _towers_run(pa, pc, x, src3d, dst4_3d, qm4, ea_t, e_pad,
                zeros_pk, deg32):
    out_a = _lin0(x, pa['lin0_W'], pa['lin0_b'])
    out_c = _lin0(x, pc['lin0_W'], pc['lin0_b'])
    we_a = _edgenet(ea_t, pa['nn1_W'], pa['nn1_b'], pa['nn2_W'], pa['nn2_b'])
    we_b = _edgenet(ea_t, pc['nn1_W'], pc['nn1_b'], pc['nn2_W'], pc['nn2_b'])
    k_chunks = e_pad // _NW // 128
    for _ in range(6):
        ga, gc = _sc_gather2(out_a, out_c, src3d, e_pad, k_chunks)
        msg_a = _edge_einsum(we_a, ga[:, :_DIM].T, qm4)
        msg_c = _edge_einsum(we_b, gc[:, :_DIM].T, qm4)
        aggp = _sc_scatter2(msg_a.T, msg_c.T, dst4_3d, zeros_pk)
        agg_a = (aggp[0, 0] + aggp[1, 0]).reshape(-1, _DIM)
        agg_c = (aggp[0, 1] + aggp[1, 1]).reshape(-1, _DIM)
        out_a = _node_update(out_a, agg_a, deg32, pa)
        out_c = _node_update(out_c, agg_c, deg32, pc)
    return out_a, out_c


def kernel(x, edge_index, edge_attr, batch, nonring, params):
    dim = _DIM
    n_nodes = x.shape[0]
    n_edges = edge_attr.shape[0]
    e_pad = ((n_edges + 4095) // 4096) * 4096
    n_pad = ((n_nodes + _NB - 1) // _NB) * _NB
    pa = params['actor']
    pc = params['critic']

    x_pad = jnp.pad(x, ((0, n_pad - n_nodes), (0, 0)))
    dst_pad = jnp.pad(edge_index[1], (0, e_pad - n_edges))
    src3d = jnp.pad(edge_index[0], (0, e_pad - n_edges)).reshape(_NW, -1, 128)
    dst4_3d = (dst_pad // 4).reshape(_NW, -1, 128)
    qm4 = (dst_pad[None, :] % 4 ==
           jnp.arange(4)[:, None]).astype(jnp.float32)
    ea_t = jnp.pad(edge_attr, ((0, e_pad - n_edges), (0, 0))).T
    mask = (jnp.arange(e_pad) < n_edges).astype(jnp.float32)
    deg_in = jnp.repeat(qm4 * mask[None, :], dim, axis=0).T
    zeros_pk = jnp.zeros((n_pad // 4, 128), jnp.float32)
    degp = _sc_scatter(deg_in, dst4_3d, zeros_pk)
    deg32 = (degp[0] + degp[1]).reshape(-1, dim)

    out_a, out_c = _towers_run(pa, pc, x_pad, src3d, dst4_3d, qm4, ea_t,
                               e_pad, zeros_pk, deg32)
    pool_a = _s2s_actor(out_a, pa, n_nodes)
    v = _s2s_critic(out_c, pc, n_nodes)

    t = nonring.shape[0]
    nr_pad = 8192
    nr3d = jnp.pad(nonring.reshape(-1),
                   (0, nr_pad - 4 * t)).reshape(_NW, -1, 128)
    sel = _sc_gather(out_a, nr3d, nr_pad, nr_pad // _NW // 128)[:4 * t, :dim]
    sel = sel.reshape(4 * dim, -1).T
    pool_rep = jnp.repeat(pool_a.reshape(-1), t).reshape(t, -1)
    feat = jnp.concatenate([sel, pool_rep], axis=-1)
    logits, logp, ent = _actor_head(feat, pa, 6)
    action = jax.random.categorical(jax.random.key(123), logits, axis=-1)
    log_prob = jnp.take_along_axis(logp, action[:, None], axis=1)[:, 0]
    return logits, action, log_prob, ent[:, 0], v


# speedup vs baseline: 1.1459x; 1.1459x over previous
import functools

import jax
import jax.numpy as jnp
from jax import lax
from jax.experimental import pallas as pl
from jax.experimental.pallas import tpu as pltpu
from jax.experimental.pallas import tpu_sc as plsc

_NW = 32


def _gather_pipeline(jobs, idx_v, bufs, gsems, osems, depth=4):
    k = len(jobs)
    gcps = [None] * k
    ocps = [None] * k

    def fire(j):
        tab, jj, _ = jobs[j]
        b = j % depth
        gcps[j] = pltpu.async_copy(tab.at[idx_v.at[jj]], bufs.at[b],
                                   gsems.at[b])

    for j in range(min(depth, k)):
        fire(j)
    for j in range(k):
        b = j % depth
        gcps[j].wait()
        ocps[j] = pltpu.async_copy(bufs.at[b], jobs[j][2], osems.at[b])
        if j + depth < k:
            ocps[j].wait()
            fire(j + depth)
    for j in range(max(0, k - depth), k):
        ocps[j].wait()


def _sc_gather(table, idx3d, n_out, k_chunks):
    rpw = k_chunks
    mesh = plsc.VectorSubcoreMesh(core_axis_name="c", subcore_axis_name="s")

    @functools.partial(
        pl.kernel, mesh=mesh,
        out_type=jax.ShapeDtypeStruct((n_out, 128), jnp.float32),
        scratch_types=[
            pltpu.VMEM((rpw, 128), jnp.int32),
            pltpu.VMEM((4, 128, 128), jnp.float32),
            pltpu.SemaphoreType.DMA((4,)),
            pltpu.SemaphoreType.DMA((4,)),
        ],
    )
    def k(table_hbm, idx_hbm, out_hbm, idx_v, bufs, gsems, osems):
        wid = lax.axis_index("s") * 2 + lax.axis_index("c")
        pltpu.sync_copy(idx_hbm.at[wid], idx_v)
        jobs = [(table_hbm, j,
                 out_hbm.at[pl.ds((wid * k_chunks + j) * 128, 128)])
                for j in range(k_chunks)]
        _gather_pipeline(jobs, idx_v, bufs, gsems, osems)

    return k(table, idx3d)


def _sc_gather2(table_a, table_b, idx3d, n_out, k_chunks):
    rpw = k_chunks
    mesh = plsc.VectorSubcoreMesh(core_axis_name="c", subcore_axis_name="s")

    @functools.partial(
        pl.kernel, mesh=mesh,
        out_type=(jax.ShapeDtypeStruct((n_out, 128), jnp.float32),
                  jax.ShapeDtypeStruct((n_out, 128), jnp.float32)),
        scratch_types=[
            pltpu.VMEM((rpw, 128), jnp.int32),
            pltpu.VMEM((4, 128, 128), jnp.float32),
            pltpu.SemaphoreType.DMA((4,)),
            pltpu.SemaphoreType.DMA((4,)),
        ],
    )
    def k(ta_hbm, tb_hbm, idx_hbm, oa_hbm, ob_hbm, idx_v, bufs,
          gsems, osems):
        wid = lax.axis_index("s") * 2 + lax.axis_index("c")
        pltpu.sync_copy(idx_hbm.at[wid], idx_v)
        jobs = []
        for tab, out in ((ta_hbm, oa_hbm), (tb_hbm, ob_hbm)):
            jobs += [(tab, j,
                      out.at[pl.ds((wid * k_chunks + j) * 128, 128)])
                     for j in range(k_chunks)]
        _gather_pipeline(jobs, idx_v, bufs, gsems, osems)

    return k(table_a, table_b, idx3d)


def _sc_scatter(msg128, idx3d, zeros_pk):
    n_pk = zeros_pk.shape[0]
    e_pad = msg128.shape[0]
    rpw = e_pad // _NW // 128
    npt = n_pk // 16
    mesh = plsc.VectorSubcoreMesh(core_axis_name="c", subcore_axis_name="s")

    @functools.partial(
        pl.kernel, mesh=mesh,
        out_type=jax.ShapeDtypeStruct((2, n_pk, 128), jnp.float32),
        scratch_types=[
            pltpu.VMEM((rpw, 128), jnp.int32),
            pltpu.VMEM((2, 128, 128), jnp.float32),
            pltpu.SemaphoreType.DMA((2,)),
            pltpu.VMEM_SHARED((n_pk, 128), jnp.float32),
        ],
    )
    def k(msg_hbm, idx_hbm, zero_hbm, out_hbm, idx_v, bufs, lsems, agg_sh):
        c = lax.axis_index("c")
        s = lax.axis_index("s")
        wid = s * 2 + c
        pltpu.sync_copy(zero_hbm.at[pl.ds(s * npt, npt)],
                        agg_sh.at[pl.ds(s * npt, npt)])
        pltpu.sync_copy(idx_hbm.at[wid], idx_v)
        plsc.subcore_barrier()
        lcps = [None] * rpw

        def fire(j):
            lcps[j] = pltpu.async_copy(
                msg_hbm.at[pl.ds((wid * rpw + j) * 128, 128)],
                bufs.at[j & 1], lsems.at[j & 1])

        fire(0)
        for j in range(rpw):
            lcps[j].wait()
            if j + 1 < rpw:
                fire(j + 1)
            pltpu.sync_copy(bufs.at[j & 1], agg_sh.at[idx_v.at[j]],
                            add=True)
        plsc.subcore_barrier()
        pltpu.sync_copy(agg_sh.at[pl.ds(s * npt, npt)],
                        out_hbm.at[c, pl.ds(s * npt, npt)])

    return k(msg128, idx3d, zeros_pk)


def _sc_scatter2(msg_a, msg_b, idx3d, zeros_pk):
    n_pk = zeros_pk.shape[0]
    e_pad = msg_a.shape[0]
    rpw = e_pad // _NW // 128
    npt = n_pk // 16
    mesh = plsc.VectorSubcoreMesh(core_axis_name="c", subcore_axis_name="s")

    @functools.partial(
        pl.kernel, mesh=mesh,
        out_type=jax.ShapeDtypeStruct((2, 2, n_pk, 128), jnp.float32),
        scratch_types=[
            pltpu.VMEM((rpw, 128), jnp.int32),
            pltpu.VMEM((2, 128, 128), jnp.float32),
            pltpu.SemaphoreType.DMA((2,)),
            pltpu.VMEM_SHARED((n_pk, 128), jnp.float32),
            pltpu.VMEM_SHARED((n_pk, 128), jnp.float32),
        ],
    )
    def k(ma_hbm, mb_hbm, idx_hbm, zero_hbm, out_hbm, idx_v, bufs,
          lsems, agg_a, agg_b):
        c = lax.axis_index("c")
        s = lax.axis_index("s")
        wid = s * 2 + c
        pltpu.sync_copy(zero_hbm.at[pl.ds(s * npt, npt)],
                        agg_a.at[pl.ds(s * npt, npt)])
        pltpu.sync_copy(zero_hbm.at[pl.ds(s * npt, npt)],
                        agg_b.at[pl.ds(s * npt, npt)])
        pltpu.sync_copy(idx_hbm.at[wid], idx_v)
        plsc.subcore_barrier()
        jobs = []
        for msg, acc in ((ma_hbm, agg_a), (mb_hbm, agg_b)):
            jobs += [(msg.at[pl.ds((wid * rpw + j) * 128, 128)], acc, j)
                     for j in range(rpw)]
        lcps = [None] * len(jobs)

        def fire(j):
            lcps[j] = pltpu.async_copy(jobs[j][0], bufs.at[j & 1],
                                       lsems.at[j & 1])

        fire(0)
        for j in range(len(jobs)):
            lcps[j].wait()
            if j + 1 < len(jobs):
                fire(j + 1)
            pltpu.sync_copy(bufs.at[j & 1],
                            jobs[j][1].at[idx_v.at[jobs[j][2]]], add=True)
        plsc.subcore_barrier()
        pltpu.sync_copy(agg_a.at[pl.ds(s * npt, npt)],
                        out_hbm.at[c, 0, pl.ds(s * npt, npt)])
        pltpu.sync_copy(agg_b.at[pl.ds(s * npt, npt)],
                        out_hbm.at[c, 1, pl.ds(s * npt, npt)])

    return k(msg_a, msg_b, idx3d, zeros_pk)

_DIM = 32
_EB = 1024
_NB = 2048
_TB = 400
_S2S_C = 2000


def _lin0_body(x_ref, w_ref, b_ref, o_ref):
    res = jnp.maximum(
        jnp.dot(x_ref[...], w_ref[...], preferred_element_type=jnp.float32)
        + b_ref[...], 0.0)
    o_ref[...] = jnp.concatenate(
        [res, jnp.zeros((res.shape[0], 128 - _DIM), jnp.float32)], axis=1)


def _lin0(x, w, b):
    n = x.shape[0]
    return pl.pallas_call(
        _lin0_body,
        grid=(n // _NB,),
        in_specs=[
            pl.BlockSpec((_NB, 3), lambda i: (i, 0)),
            pl.BlockSpec((3, _DIM), lambda i: (0, 0)),
            pl.BlockSpec((1, _DIM), lambda i: (0, 0)),
        ],
        out_specs=pl.BlockSpec((_NB, 128), lambda i: (i, 0)),
        out_shape=jax.ShapeDtypeStruct((n, 128), jnp.float32),
    )(x, w, b[None, :])


def _edgenet_body(ea_ref, w1_ref, b1_ref, w2_ref, b2_ref, o_ref):
    h1 = jnp.maximum(
        jnp.dot(w1_ref[...], ea_ref[...], preferred_element_type=jnp.float32)
        + b1_ref[...], 0.0)
    o_ref[...] = (
        jnp.dot(w2_ref[...], h1, preferred_element_type=jnp.float32)
        + b2_ref[...])


def _edgenet(ea_t, nn1_W, nn1_b, nn2_W, nn2_b):
    e_pad = ea_t.shape[1]
    c = _DIM * _DIM
    return pl.pallas_call(
        _edgenet_body,
        grid=(e_pad // _EB,),
        in_specs=[
            pl.BlockSpec((7, _EB), lambda i: (0, i)),
            pl.BlockSpec((128, 7), lambda i: (0, 0)),
            pl.BlockSpec((128, 1), lambda i: (0, 0)),
            pl.BlockSpec((c, 128), lambda i: (0, 0)),
            pl.BlockSpec((c, 1), lambda i: (0, 0)),
        ],
        out_specs=pl.BlockSpec((c, _EB), lambda i: (0, i)),
        out_shape=jax.ShapeDtypeStruct((c, e_pad), jnp.float32),
        compiler_params=pltpu.CompilerParams(
            dimension_semantics=("arbitrary",)),
    )(ea_t, nn1_W.T, nn1_b[:, None], nn2_W.T, nn2_b[:, None])


def _einsum_body(we_ref, a_ref, qm_ref, o_ref):
    a = a_ref[...]
    acc = we_ref[pl.ds(0, _DIM), :] * a[0:1, :]
    for d in range(1, _DIM):
        acc = acc + we_ref[pl.ds(d * _DIM, _DIM), :] * a[d:d + 1, :]
    qm = qm_ref[...]
    o_ref[...] = jnp.concatenate(
        [acc * qm[q:q + 1, :] for q in range(4)], axis=0)


def _edge_einsum(we_c, a_t, qm4):
    e_pad = we_c.shape[1]
    c = _DIM * _DIM
    return pl.pallas_call(
        _einsum_body,
        grid=(e_pad // _EB,),
        in_specs=[
            pl.BlockSpec((c, _EB), lambda i: (0, i)),
            pl.BlockSpec((_DIM, _EB), lambda i: (0, i)),
            pl.BlockSpec((4, _EB), lambda i: (0, i)),
        ],
        out_specs=pl.BlockSpec((128, _EB), lambda i: (0, i)),
        out_shape=jax.ShapeDtypeStruct((128, e_pad), jnp.float32),
        compiler_params=pltpu.CompilerParams(
            dimension_semantics=("arbitrary",)),
    )(we_c, a_t, qm4)


def _node_body(out_ref, agg_ref, deg_ref, wc_ref, bc_ref,
               wir_ref, wiz_ref, win_ref, whr_ref, whz_ref, whn_ref,
               o_ref):
    out = out_ref[...][:, :_DIM]
    aggn = agg_ref[...] / jnp.maximum(deg_ref[...], 1.0)
    dot = functools.partial(jnp.dot, preferred_element_type=jnp.float32)
    m = jnp.maximum(dot(out, wc_ref[...]) + aggn + bc_ref[...], 0.0)
    r = jax.nn.sigmoid(dot(m, wir_ref[...]) + dot(out, whr_ref[...]))
    z = jax.nn.sigmoid(dot(m, wiz_ref[...]) + dot(out, whz_ref[...]))
    n = jnp.tanh(dot(m, win_ref[...]) + r * dot(out, whn_ref[...]))
    new = (1.0 - z) * n + z * out
    o_ref[...] = jnp.concatenate(
        [new, jnp.zeros((new.shape[0], 128 - _DIM), jnp.float32)], axis=1)


def _node_update(out, agg, deg32, p):
    n = out.shape[0]
    wih = p['gru_Wih']
    whh = p['gru_Whh']
    nb128 = pl.BlockSpec((_NB, 128), lambda i: (i, 0))
    nb = pl.BlockSpec((_NB, _DIM), lambda i: (i, 0))
    wb = pl.BlockSpec((_DIM, _DIM), lambda i: (0, 0))
    return pl.pallas_call(
        _node_body,
        grid=(n // _NB,),
        in_specs=[nb128, nb, nb, wb,
                  pl.BlockSpec((1, _DIM), lambda i: (0, 0)),
                  wb, wb, wb, wb, wb, wb],
        out_specs=nb128,
        out_shape=jax.ShapeDtypeStruct((n, 128), jnp.float32),
    )(out, agg, deg32, p['conv_root'], p['conv_b'][None, :],
      wih[:, :_DIM], wih[:, _DIM:2 * _DIM], wih[:, 2 * _DIM:],
      whh[:, :_DIM], whh[:, _DIM:2 * _DIM], whh[:, 2 * _DIM:])


def _s2s_loop(out_ref, wih_ref, whh_ref, bih_ref, bhh_ref, n_real):
    dot = functools.partial(jnp.dot, preferred_element_type=jnp.float32)
    nch = n_real // _S2S_C
    q_star = jnp.zeros((1, 2 * _DIM), jnp.float32)
    hs = jnp.zeros((1, _DIM), jnp.float32)
    cs = jnp.zeros((1, _DIM), jnp.float32)
    for _ in range(6):
        g = (dot(q_star, wih_ref[...]) + bih_ref[...]
             + dot(hs, whh_ref[...]) + bhh_ref[...])
        i = jax.nn.sigmoid(g[:, :_DIM])
        f = jax.nn.sigmoid(g[:, _DIM:2 * _DIM])
        gg = jnp.tanh(g[:, 2 * _DIM:3 * _DIM])
        o = jax.nn.sigmoid(g[:, 3 * _DIM:])
        cs = f * cs + i * gg
        hs = o * jnp.tanh(cs)

        def _emax_body(k, m):
            ch = out_ref[pl.ds(k * _S2S_C, _S2S_C), :][:, :_DIM]
            e = jnp.sum(ch * hs, axis=1, keepdims=True)
            return jnp.maximum(m, jnp.max(e))

        emax = lax.fori_loop(0, nch, _emax_body, jnp.float32(-jnp.inf))

        def _acc_body(k, carry):
            den, racc = carry
            ch = out_ref[pl.ds(k * _S2S_C, _S2S_C), :][:, :_DIM]
            e = jnp.sum(ch * hs, axis=1, keepdims=True)
            ex = jnp.exp(e - emax)
            return (den + jnp.sum(ex),
                    racc + jnp.sum(ex * ch, axis=0, keepdims=True))

        den, racc = lax.fori_loop(
            0, nch, _acc_body,
            (jnp.float32(0.0), jnp.zeros((1, _DIM), jnp.float32)))
        q_star = jnp.concatenate([hs, racc / den], axis=1)
    return q_star


def _s2s_actor(out, p, n_real):
    def body(out_ref, wih_ref, whh_ref, bih_ref, bhh_ref, q_ref):
        q_ref[...] = _s2s_loop(out_ref, wih_ref, whh_ref, bih_ref,
                               bhh_ref, n_real)

    return pl.pallas_call(
        body,
        out_shape=jax.ShapeDtypeStruct((1, 2 * _DIM), jnp.float32),
    )(out, p['s2s_Wih'], p['s2s_Whh'], p['s2s_bih'][None, :],
      p['s2s_bhh'][None, :])


def _s2s_critic_body(n_real, out_ref, wih_ref, whh_ref, bih_ref, bhh_ref,
                     mwih_ref, mb_ref, l1_ref, l1b_ref, l3_ref, l3b_ref,
                     v_ref):
    q_star = _s2s_loop(out_ref, wih_ref, whh_ref, bih_ref, bhh_ref, n_real)
    dot = functools.partial(jnp.dot, preferred_element_type=jnp.float32)
    hid = 2 * _DIM
    g = dot(q_star, mwih_ref[...]) + mb_ref[...]
    i = jax.nn.sigmoid(g[:, :hid])
    gg = jnp.tanh(g[:, 2 * hid:3 * hid])
    o = jax.nn.sigmoid(g[:, 3 * hid:])
    hv = o * jnp.tanh(i * gg)
    oc = jnp.maximum(dot(hv, l1_ref[...]) + l1b_ref[...], 0.0)
    v_ref[...] = dot(oc, l3_ref[...]) + l3b_ref[...]


def _s2s_critic(out, p, n_real):
    return pl.pallas_call(
        functools.partial(_s2s_critic_body, n_real),
        out_shape=jax.ShapeDtypeStruct((1, 1), jnp.float32),
    )(out, p['s2s_Wih'], p['s2s_Whh'], p['s2s_bih'][None, :],
      p['s2s_bhh'][None, :],
      p['mem_Wih'], (p['mem_bih'] + p['mem_bhh'])[None, :],
      p['lin1_W'], p['lin1_b'][None, :], p['lin3_W'], p['lin3_b'][None, :])


def _head_body(feat_ref, wi_ref, wg_ref, wo_ref, bi_ref, bg_ref, bo_ref,
               l1_ref, l1b_ref, l2_ref, l2b_ref,
               logits_ref, logp_ref, ent_ref):
    dot = functools.partial(jnp.dot, preferred_element_type=jnp.float32)
    feat = feat_ref[...]
    i = jax.nn.sigmoid(dot(feat, wi_ref[...]) + bi_ref[...])
    gg = jnp.tanh(dot(feat, wg_ref[...]) + bg_ref[...])
    o = jax.nn.sigmoid(dot(feat, wo_ref[...]) + bo_ref[...])
    hm = o * jnp.tanh(i * gg)
    o1 = jnp.maximum(dot(hm, l1_ref[...]) + l1b_ref[...], 0.0)
    logits = dot(o1, l2_ref[...]) + l2b_ref[...]
    m = jnp.max(logits, axis=1, keepdims=True)
    ex = jnp.exp(logits - m)
    lse = jnp.log(jnp.sum(ex, axis=1, keepdims=True)) + m
    logp = logits - lse
    logits_ref[...] = logits
    logp_ref[...] = logp
    ent_ref[...] = -jnp.sum(jnp.exp(logp) * logp, axis=1, keepdims=True)


def _actor_head(feat, p, action_dim):
    t = feat.shape[0]
    hid = 6 * _DIM
    wih = p['mem_Wih']
    b = p['mem_bih'] + p['mem_bhh']
    fb = pl.BlockSpec((_TB, hid), lambda i: (i, 0))
    wb = pl.BlockSpec((hid, hid), lambda i: (0, 0))
    bb = pl.BlockSpec((1, hid), lambda i: (0, 0))
    ob = pl.BlockSpec((_TB, action_dim), lambda i: (i, 0))
    return pl.pallas_call(
        _head_body,
        grid=(t // _TB,),
        in_specs=[fb, wb, wb, wb, bb, bb, bb,
                  pl.BlockSpec((hid, 2 * _DIM), lambda i: (0, 0)),
                  pl.BlockSpec((1, 2 * _DIM), lambda i: (0, 0)),
                  pl.BlockSpec((2 * _DIM, action_dim), lambda i: (0, 0)),
                  pl.BlockSpec((1, action_dim), lambda i: (0, 0))],
        out_specs=(ob, ob, pl.BlockSpec((_TB, 1), lambda i: (i, 0))),
        out_shape=(
            jax.ShapeDtypeStruct((t, action_dim), jnp.float32),
            jax.ShapeDtypeStruct((t, action_dim), jnp.float32),
            jax.ShapeDtypeStruct((t, 1), jnp.float32),
        ),
    )(feat, wih[:, :hid], wih[:, 2 * hid:3 * hid], wih[:, 3 * hid:],
      b[None, :hid], b[None, 2 * hid:3 * hid], b[None, 3 * hid:],
      p['lin1_W'], p['lin1_b'][None, :], p['lin2_W'], p['lin2_b'][None, :])


def _towers_run(pa, pc, x, src3d, dst4_3d, qm4, ea_t, e_pad,
                zeros_pk, deg32):
    out_a = _lin0(x, pa['lin0_W'], pa['lin0_b'])
    out_c = _lin0(x, pc['lin0_W'], pc['lin0_b'])
    we_a = _edgenet(ea_t, pa['nn1_W'], pa['nn1_b'], pa['nn2_W'], pa['nn2_b'])
    we_b = _edgenet(ea_t, pc['nn1_W'], pc['nn1_b'], pc['nn2_W'], pc['nn2_b'])
    k_chunks = e_pad // _NW // 128
    for _ in range(6):
        ga = _sc_gather(out_a, src3d, e_pad, k_chunks)
        gc = _sc_gather(out_c, src3d, e_pad, k_chunks)
        msg_a = _edge_einsum(we_a, ga[:, :_DIM].T, qm4)
        msg_c = _edge_einsum(we_b, gc[:, :_DIM].T, qm4)
        apa = _sc_scatter(msg_a.T, dst4_3d, zeros_pk)
        apc = _sc_scatter(msg_c.T, dst4_3d, zeros_pk)
        agg_a = (apa[0] + apa[1]).reshape(-1, _DIM)
        agg_c = (apc[0] + apc[1]).reshape(-1, _DIM)
        out_a = _node_update(out_a, agg_a, deg32, pa)
        out_c = _node_update(out_c, agg_c, deg32, pc)
    return out_a, out_c


def kernel(x, edge_index, edge_attr, batch, nonring, params):
    dim = _DIM
    n_nodes = x.shape[0]
    n_edges = edge_attr.shape[0]
    e_pad = ((n_edges + 4095) // 4096) * 4096
    n_pad = ((n_nodes + _NB - 1) // _NB) * _NB
    pa = params['actor']
    pc = params['critic']

    x_pad = jnp.pad(x, ((0, n_pad - n_nodes), (0, 0)))
    dst_pad = jnp.pad(edge_index[1], (0, e_pad - n_edges))
    src3d = jnp.pad(edge_index[0], (0, e_pad - n_edges)).reshape(_NW, -1, 128)
    dst4_3d = (dst_pad // 4).reshape(_NW, -1, 128)
    qm4 = (dst_pad[None, :] % 4 ==
           jnp.arange(4)[:, None]).astype(jnp.float32)
    ea_t = jnp.pad(edge_attr, ((0, e_pad - n_edges), (0, 0))).T
    mask = (jnp.arange(e_pad) < n_edges).astype(jnp.float32)
    deg_in = jnp.repeat(qm4 * mask[None, :], dim, axis=0).T
    zeros_pk = jnp.zeros((n_pad // 4, 128), jnp.float32)
    degp = _sc_scatter(deg_in, dst4_3d, zeros_pk)
    deg32 = (degp[0] + degp[1]).reshape(-1, dim)

    out_a, out_c = _towers_run(pa, pc, x_pad, src3d, dst4_3d, qm4, ea_t,
                               e_pad, zeros_pk, deg32)
    pool_a = _s2s_actor(out_a, pa, n_nodes)
    v = _s2s_critic(out_c, pc, n_nodes)

    t = nonring.shape[0]
    nr_pad = 8192
    nr3d = jnp.pad(nonring.reshape(-1),
                   (0, nr_pad - 4 * t)).reshape(_NW, -1, 128)
    sel = _sc_gather(out_a, nr3d, nr_pad, nr_pad // _NW // 128)[:4 * t, :dim]
    sel = sel.reshape(4 * dim, -1).T
    pool_rep = jnp.repeat(pool_a.reshape(-1), t).reshape(t, -1)
    feat = jnp.concatenate([sel, pool_rep], axis=-1)
    logits, logp, ent = _actor_head(feat, pa, 6)
    action = jax.random.categorical(jax.random.key(123), logits, axis=-1)
    log_prob = jnp.take_along_axis(logp, action[:, None], axis=1)[:, 0]
    return logits, action, log_prob, ent[:, 0], v
